# baseline (speedup 1.0000x reference)
import functools

import jax
import jax.numpy as jnp
from jax import lax
from jax.experimental import pallas as pl
from jax.experimental.pallas import tpu as pltpu
from jax.experimental.pallas import tpu_sc as plsc

_B, _S, _D = 4, 2048, 1024
_NC, _NS = 2, 16
_NW = _NC * _NS
_PPW = _S // _NW
_C = 8
_NPC = _PPW // _C
_NCH = _NPC * _B
_NBUF = 4
_AHEAD = 3

_mesh = plsc.VectorSubcoreMesh(core_axis_name="c", subcore_axis_name="s")


@functools.partial(
    pl.kernel,
    out_type=jax.ShapeDtypeStruct((_B * _S, _D), jnp.float32),
    mesh=_mesh,
    scratch_types=[
        pltpu.VMEM((_B, _NPC, _C), jnp.int32),
        pltpu.VMEM((_C, _D), jnp.float32),
        pltpu.VMEM((_C, _D), jnp.float32),
        pltpu.VMEM((_C, _D), jnp.float32),
        pltpu.VMEM((_C, _D), jnp.float32),
        pltpu.VMEM((_PPW, _D), jnp.float32),
        pltpu.SemaphoreType.DMA,
        pltpu.SemaphoreType.DMA,
        pltpu.SemaphoreType.DMA,
        pltpu.SemaphoreType.DMA,
        pltpu.SemaphoreType.DMA,
        pltpu.SemaphoreType.DMA,
        pltpu.SemaphoreType.DMA,
        pltpu.SemaphoreType.DMA,
        pltpu.SemaphoreType.DMA,
    ],
)
def _emb_lookup(tok_idx, tok_tab, pos_tab, out,
                idx_v, buf0, buf1, buf2, buf3, pos_v,
                g0, g1, g2, g3, o0, o1, o2, o3, psem):
    wid = lax.axis_index("s") * _NC + lax.axis_index("c")
    pos0 = wid * _PPW
    pd = pltpu.async_copy(pos_tab.at[pl.ds(pos0, _PPW)], pos_v, psem)
    pltpu.sync_copy(tok_idx.at[wid], idx_v)
    bufs = (buf0, buf1, buf2, buf3)
    gsems = (g0, g1, g2, g3)
    osems = (o0, o1, o2, o3)

    def gissue(ci, q):
        pltpu.async_copy(
            tok_tab.at[idx_v.at[ci % _B, ci // _B]], bufs[q], gsems[q])

    def gwait(q):
        pltpu.make_async_copy(tok_tab.at[idx_v.at[0, 0]], bufs[q],
                              gsems[q]).wait()

    def oissue(ci, q):
        pltpu.async_copy(
            bufs[q],
            out.at[pl.ds((ci % _B) * _S + pos0 + (ci // _B) * _C, _C)],
            osems[q])

    def owait(q):
        pltpu.make_async_copy(bufs[q], out.at[pl.ds(0, _C)],
                              osems[q]).wait()

    for ci in range(_AHEAD):
        gissue(ci, ci % _NBUF)
    pd.wait()

    def group(gi, carry):
        ci0 = gi * _NBUF
        for p in range(_NBUF):
            ci = ci0 + p
            gwait(p)
            buf = bufs[p]
            prow = (ci // _B) * _C

            @plsc.parallel_loop(0, _C, unroll=2)
            def _add_row(r):
                for j in range(_D // 16):
                    plsc.addupdate(buf.at[r, pl.ds(j * 16, 16)],
                                   pos_v[prow + r, pl.ds(j * 16, 16)])

            oissue(ci, p)
            nci = ci + _AHEAD
            q = (p + _AHEAD) % _NBUF

            @pl.when(jnp.logical_and(nci < _NCH, nci >= _NBUF))
            def _():
                owait(q)
                gissue(nci, q)

            @pl.when(jnp.logical_and(nci < _NCH, nci < _NBUF))
            def _():
                gissue(nci, q)

        return carry

    lax.fori_loop(0, _NCH // _NBUF, group, 0)
    for p in range(_NBUF):
        owait(p)


def kernel(x, token_table, pos_table):
    B, S = x.shape
    D = token_table.shape[1]
    tok_idx = (x.reshape(B, _NW, _NPC, _C).astype(jnp.int32)
               .transpose(1, 0, 2, 3))
    out = _emb_lookup(tok_idx, token_table, pos_table)
    return out.reshape(B, S, D)


# speedup vs baseline: 1.0394x; 1.0394x over previous
import functools

import jax
import jax.numpy as jnp
from jax import lax
from jax.experimental import pallas as pl
from jax.experimental.pallas import tpu as pltpu
from jax.experimental.pallas import tpu_sc as plsc

_B, _S, _D = 4, 2048, 1024
_NC, _NS = 2, 16
_NW = _NC * _NS
_PPW = _S // _NW
_C = 16
_NPC = _PPW // _C
_NCH = _NPC * _B
_NBUF = 4
_AHEAD = 3
_HALF = 2 * _C

_mesh = plsc.VectorSubcoreMesh(core_axis_name="c", subcore_axis_name="s")


@functools.partial(
    pl.kernel,
    out_type=jax.ShapeDtypeStruct((_B * _S, _D), jnp.float32),
    mesh=_mesh,
    scratch_types=[
        pltpu.VMEM((_B, _NPC, _C), jnp.int32),
        pltpu.VMEM((_C, _D), jnp.float32),
        pltpu.VMEM((_C, _D), jnp.float32),
        pltpu.VMEM((_C, _D), jnp.float32),
        pltpu.VMEM((_C, _D), jnp.float32),
        pltpu.VMEM((_HALF, _D), jnp.float32),
        pltpu.SemaphoreType.DMA,
        pltpu.SemaphoreType.DMA,
        pltpu.SemaphoreType.DMA,
        pltpu.SemaphoreType.DMA,
        pltpu.SemaphoreType.DMA,
        pltpu.SemaphoreType.DMA,
        pltpu.SemaphoreType.DMA,
        pltpu.SemaphoreType.DMA,
        pltpu.SemaphoreType.DMA,
    ],
)
def _emb_lookup(tok_idx, tok_tab, pos_tab, out,
                idx_v, buf0, buf1, buf2, buf3, pos_v,
                g0, g1, g2, g3, o0, o1, o2, o3, psem):
    wid = lax.axis_index("s") * _NC + lax.axis_index("c")
    pos0 = wid * _PPW
    pd = pltpu.async_copy(pos_tab.at[pl.ds(pos0, _HALF)], pos_v, psem)
    for b in range(_B):
        pltpu.sync_copy(tok_idx.at[b, wid], idx_v.at[b])
    bufs = (buf0, buf1, buf2, buf3)
    gsems = (g0, g1, g2, g3)
    osems = (o0, o1, o2, o3)

    def gissue(ci, q):
        pltpu.async_copy(
            tok_tab.at[idx_v.at[ci % _B, ci // _B]], bufs[q], gsems[q])

    def gwait(q):
        pltpu.make_async_copy(tok_tab.at[idx_v.at[0, 0]], bufs[q],
                              gsems[q]).wait()

    def oissue(ci, q):
        pltpu.async_copy(
            bufs[q],
            out.at[pl.ds((ci % _B) * _S + pos0 + (ci // _B) * _C, _C)],
            osems[q])

    def owait(q):
        pltpu.make_async_copy(bufs[q], out.at[pl.ds(0, _C)],
                              osems[q]).wait()

    for ci in range(_AHEAD):
        gissue(ci, ci % _NBUF)
    pd.wait()

    def group(gi, carry):
        ci0 = gi * _NBUF
        for p in range(_NBUF):
            ci = ci0 + p
            if p == 0:
                @pl.when(ci == _NCH // 2)
                def _():
                    pltpu.make_async_copy(
                        pos_tab.at[pl.ds(pos0 + _HALF, _HALF)], pos_v,
                        psem).wait()

            gwait(p)
            buf = bufs[p]
            prow = ((ci // _B) % 2) * _C

            @plsc.parallel_loop(0, _C, unroll=2)
            def _add_row(r):
                for j in range(_D // 16):
                    plsc.addupdate(buf.at[r, pl.ds(j * 16, 16)],
                                   pos_v[prow + r, pl.ds(j * 16, 16)])

            oissue(ci, p)
            if p == _NBUF - 1:
                @pl.when(ci == _NCH // 2 - 1)
                def _():
                    pltpu.async_copy(
                        pos_tab.at[pl.ds(pos0 + _HALF, _HALF)], pos_v,
                        psem)

            nci = ci + _AHEAD
            q = (p + _AHEAD) % _NBUF

            @pl.when(jnp.logical_and(nci < _NCH, nci >= _NBUF))
            def _():
                owait(q)
                gissue(nci, q)

            @pl.when(jnp.logical_and(nci < _NCH, nci < _NBUF))
            def _():
                gissue(nci, q)

        return carry

    lax.fori_loop(0, _NCH // _NBUF, group, 0)
    for p in range(_NBUF):
        owait(p)


def kernel(x, token_table, pos_table):
    B, S = x.shape
    D = token_table.shape[1]
    tok_idx = x.reshape(B, _NW, _NPC, _C).astype(jnp.int32)
    out = _emb_lookup(tok_idx, token_table, pos_table)
    return out.reshape(B, S, D)


# speedup vs baseline: 1.2804x; 1.2318x over previous
import functools

import jax
import jax.numpy as jnp
from jax import lax
from jax.experimental import pallas as pl
from jax.experimental.pallas import tpu as pltpu
from jax.experimental.pallas import tpu_sc as plsc

_B, _S, _D = 4, 2048, 1024
_NC, _NS = 2, 16
_NW = _NC * _NS
_PPW = _S // _NW
_C = 8
_NPC = _PPW // _C
_NCH = _NPC * _B
_NBUF = 4
_AHEAD = 2
_GRP = 4
_NV = _D // 16

_mesh = plsc.VectorSubcoreMesh(core_axis_name="c", subcore_axis_name="s")


@functools.partial(
    pl.kernel,
    out_type=jax.ShapeDtypeStruct((_B * _S, _D), jnp.float32),
    mesh=_mesh,
    scratch_types=[
        pltpu.VMEM((_B, _NPC, _C), jnp.int32),
        pltpu.VMEM((_PPW, _D), jnp.float32),
        pltpu.VMEM((_C, _D), jnp.float32),
        pltpu.VMEM((_C, _D), jnp.float32),
        pltpu.VMEM((_C, _D), jnp.float32),
        pltpu.VMEM((_C, _D), jnp.float32),
        pltpu.SemaphoreType.DMA,
        pltpu.SemaphoreType.DMA,
        pltpu.SemaphoreType.DMA,
        pltpu.SemaphoreType.DMA,
        pltpu.SemaphoreType.DMA,
        pltpu.SemaphoreType.DMA,
        pltpu.SemaphoreType.DMA,
        pltpu.SemaphoreType.DMA,
    ],
)
def _emb_lookup(tok_idx, tok_tab, pos_tab, out,
                idx_v, pos_v, buf0, buf1, buf2, buf3,
                g0, g1, g2, g3, o0, o1, o2, o3):
    sid = lax.axis_index("s")
    wid = sid * _NC + lax.axis_index("c")
    pos0 = wid * _PPW
    for b in range(_B):
        pltpu.sync_copy(tok_idx.at[b, wid], idx_v.at[b])
    pltpu.sync_copy(pos_tab.at[pl.ds(pos0, _PPW)], pos_v)
    bufs = (buf0, buf1, buf2, buf3)
    gsems = (g0, g1, g2, g3)
    osems = (o0, o1, o2, o3)

    def gissue(ci, q):
        pltpu.async_copy(
            tok_tab.at[idx_v.at[ci % _B, ci // _B]], bufs[q], gsems[q])

    def gwait(q):
        pltpu.make_async_copy(tok_tab.at[idx_v.at[0, 0]], bufs[q],
                              gsems[q]).wait()

    def oissue(ci, q):
        pltpu.async_copy(
            bufs[q],
            out.at[pl.ds((ci % _B) * _S + pos0 + (ci // _B) * _C, _C)],
            osems[q])

    def owait(q):
        pltpu.make_async_copy(bufs[q], out.at[pl.ds(0, _C)],
                              osems[q]).wait()

    for ci in range(_AHEAD):
        gissue(ci, ci)

    def chunk(ci, p):
        gwait(p)
        prow0 = (ci // _B) * _C

        @plsc.parallel_loop(0, _NV)
        def _(i):
            off = i * 16
            for r in range(_C):
                bufs[p][r, pl.ds(off, 16)] = (
                    bufs[p][r, pl.ds(off, 16)]
                    + pos_v[prow0 + r, pl.ds(off, 16)])

        oissue(ci, p)
        nci = ci + _AHEAD
        nq = (p + _AHEAD) % _NBUF

        @pl.when(jnp.logical_and(nci < _NCH, nci >= _NBUF))
        def _():
            owait(nq)

        @pl.when(nci < _NCH)
        def _():
            gissue(nci, nq)

    def group(gi, carry):
        ci0 = gi * _GRP
        for j in range(_GRP):
            chunk(ci0 + j, j % _NBUF)
        return carry

    lax.fori_loop(0, _NCH // _GRP, group, 0)
    for q in range(_NBUF):
        owait(q)


def kernel(x, token_table, pos_table):
    B, S = x.shape
    D = token_table.shape[1]
    tok_idx = x.reshape(B, _NW, _NPC, _C).astype(jnp.int32)
    out = _emb_lookup(tok_idx, token_table, pos_table)
    return out.reshape(B, S, D)


# speedup vs baseline: 1.2981x; 1.0139x over previous
import functools

import jax
import jax.numpy as jnp
from jax import lax
from jax.experimental import pallas as pl
from jax.experimental.pallas import tpu as pltpu
from jax.experimental.pallas import tpu_sc as plsc

_B, _S, _D = 4, 2048, 1024
_NC, _NS = 2, 16
_NW = _NC * _NS
_PPW = _S // _NW
_C = 8
_NPC = _PPW // _C
_NCH = _NPC * _B
_NBUF = 4
_AHEAD = 2
_GRP = 4
_NV = _D // 16

_mesh = plsc.VectorSubcoreMesh(core_axis_name="c", subcore_axis_name="s")


@functools.partial(
    pl.kernel,
    out_type=jax.ShapeDtypeStruct((_B * _S, _D), jnp.float32),
    mesh=_mesh,
    scratch_types=[
        pltpu.VMEM((_B, _NPC, _C), jnp.int32),
        pltpu.VMEM((_PPW, _D), jnp.float32),
        pltpu.VMEM((_C, _D), jnp.float32),
        pltpu.VMEM((_C, _D), jnp.float32),
        pltpu.VMEM((_C, _D), jnp.float32),
        pltpu.VMEM((_C, _D), jnp.float32),
        pltpu.SemaphoreType.DMA,
        pltpu.SemaphoreType.DMA,
        pltpu.SemaphoreType.DMA,
        pltpu.SemaphoreType.DMA,
        pltpu.SemaphoreType.DMA,
        pltpu.SemaphoreType.DMA,
        pltpu.SemaphoreType.DMA,
        pltpu.SemaphoreType.DMA,
        pltpu.SemaphoreType.DMA,
    ],
)
def _emb_lookup(tok_idx, tok_tab, pos_tab, out,
                idx_v, pos_v, buf0, buf1, buf2, buf3,
                g0, g1, g2, g3, o0, o1, o2, o3, psem):
    sid = lax.axis_index("s")
    wid = sid * _NC + lax.axis_index("c")
    pos0 = wid * _PPW
    for b in range(_B):
        pltpu.sync_copy(tok_idx.at[b, wid], idx_v.at[b])
    bufs = (buf0, buf1, buf2, buf3)
    gsems = (g0, g1, g2, g3)
    osems = (o0, o1, o2, o3)

    def gissue(ci, q):
        pltpu.async_copy(
            tok_tab.at[idx_v.at[ci % _B, ci // _B]], bufs[q], gsems[q])

    def gwait(q):
        pltpu.make_async_copy(tok_tab.at[idx_v.at[0, 0]], bufs[q],
                              gsems[q]).wait()

    def oissue(ci, q):
        pltpu.async_copy(
            bufs[q],
            out.at[pl.ds((ci % _B) * _S + pos0 + (ci // _B) * _C, _C)],
            osems[q])

    def owait(q):
        pltpu.make_async_copy(bufs[q], out.at[pl.ds(0, _C)],
                              osems[q]).wait()

    for ci in range(_AHEAD):
        gissue(ci, ci)
    pltpu.async_copy(pos_tab.at[pl.ds(pos0, _PPW)], pos_v, psem)
    pltpu.make_async_copy(pos_tab.at[pl.ds(pos0, _PPW)], pos_v, psem).wait()

    def chunk(ci, p):
        gwait(p)
        prow0 = (ci // _B) * _C

        @plsc.parallel_loop(0, _NV)
        def _(i):
            off = i * 16
            for r in range(_C):
                bufs[p][r, pl.ds(off, 16)] = (
                    bufs[p][r, pl.ds(off, 16)]
                    + pos_v[prow0 + r, pl.ds(off, 16)])

        oissue(ci, p)
        nci = ci + _AHEAD
        nq = (p + _AHEAD) % _NBUF

        @pl.when(jnp.logical_and(nci < _NCH, nci >= _NBUF))
        def _():
            owait(nq)

        @pl.when(nci < _NCH)
        def _():
            gissue(nci, nq)

    def group(gi, carry):
        ci0 = gi * _GRP
        for j in range(_GRP):
            chunk(ci0 + j, j % _NBUF)
        return carry

    lax.fori_loop(0, _NCH // _GRP, group, 0)
    for q in range(_NBUF):
        owait(q)


def kernel(x, token_table, pos_table):
    B, S = x.shape
    D = token_table.shape[1]
    tok_idx = x.reshape(B, _NW, _NPC, _C).astype(jnp.int32)
    out = _emb_lookup(tok_idx, token_table, pos_table)
    return out.reshape(B, S, D)
